# baseline (speedup 1.0000x reference)
import functools

import jax
import jax.numpy as jnp
from jax import lax
from jax.experimental import pallas as pl
from jax.experimental.pallas import tpu as pltpu
from jax.experimental.pallas import tpu_sc as plsc

B = 1024
N = 8192
K = 256
NV = N // 16
NC, NS, L = 2, 16, 16
NW = NC * NS
ROWS_PER_W = B // NW
CAP = N + 16

_MESH = plsc.VectorSubcoreMesh(
    core_axis_name="c", subcore_axis_name="s", num_cores=NC, num_subcores=NS)

def _suffix_scan(h0, h1):
    c1 = plsc.cumsum(lax.rev(h1, (0,)))
    s1 = lax.rev(c1, (0,))
    t1 = jnp.max(c1)
    c0 = plsc.cumsum(lax.rev(h0, (0,)))
    s0 = lax.rev(c0, (0,)) + t1
    return s0, s1


@functools.partial(
    pl.kernel,
    out_type=jax.ShapeDtypeStruct((B, 2 * K), jnp.float32),
    mesh=_MESH,
    compiler_params=pltpu.CompilerParams(needs_layout_passes=False),
    scratch_types=[
        pltpu.VMEM((2 * N,), jnp.float32),
        pltpu.VMEM((N,), jnp.int32),
        pltpu.VMEM((32,), jnp.int32),
        pltpu.VMEM((CAP,), jnp.int32),
        pltpu.VMEM((CAP,), jnp.int32),
        pltpu.VMEM((CAP,), jnp.int32),
        pltpu.VMEM((CAP,), jnp.int32),
        pltpu.VMEM((2 * K,), jnp.float32),
    ],
)
def _topk_sc(dgm_hbm, out_hbm, dgm_v, key_v, hist, ck0, ci0, ck1, ci1, outv):
    wid = lax.axis_index("s") * NC + lax.axis_index("c")
    iota = lax.iota(jnp.int32, L)
    zeros16 = jnp.zeros((L,), jnp.int32)
    ones16 = jnp.ones((L,), jnp.int32)

    def do_row(r, _):
        row = wid * ROWS_PER_W + r
        pltpu.sync_copy(dgm_hbm.at[row], dgm_v)

        hist[pl.ds(0, 16)] = zeros16
        hist[pl.ds(16, 16)] = zeros16

        def p1(i, _c):
            base = i * L
            rows16 = (base + iota) * 2
            bb = plsc.load_gather(dgm_v, [rows16])
            dd = plsc.load_gather(dgm_v, [rows16 + 1])
            p = dd - bb
            kb = plsc.bitcast(p, jnp.int32)
            key = kb ^ ((kb >> 31) | jnp.int32(-2**31))
            key_v[pl.ds(base, 16)] = key
            dig = (key >> 27) & 31
            plsc.addupdate_scatter(hist, [dig], ones16)
            return 0

        lax.fori_loop(0, NV, p1, 0)

        h0 = hist[pl.ds(0, 16)]
        h1 = hist[pl.ds(16, 16)]
        s0, s1 = _suffix_scan(h0, h1)
        d0 = jnp.max(jnp.where(s0 >= K, iota, -1))
        d1 = jnp.max(jnp.where(s1 >= K, iota + 16, -1))
        dstar = jnp.maximum(d0, d1)

        def p2(i, off):
            base = i * L
            key = key_v[pl.ds(base, 16)]
            dig = (key >> 27) & 31
            msk = dig >= dstar
            plsc.store_compressed(ck0.at[pl.ds(off, 16)], key, mask=msk)
            plsc.store_compressed(ci0.at[pl.ds(off, 16)], base + iota, mask=msk)
            return off + jnp.max(plsc.all_reduce_population_count(msk))

        m = lax.fori_loop(0, NV, p2, jnp.int32(0))
        ck0[pl.ds(m, 16)] = zeros16
        ci0[pl.ds(m, 16)] = zeros16
        trips = (m + 15) >> 4

        bufs = ((ck0, ci0), (ck1, ci1))
        for p in range(7):
            sk, si = bufs[p % 2]
            dk, di = bufs[(p + 1) % 2]
            sh = 5 * p

            hist[pl.ds(0, 16)] = zeros16
            hist[pl.ds(16, 16)] = zeros16

            def hcount(i, _c, sk=sk, sh=sh):
                key = sk[pl.ds(i * L, 16)]
                dig = (key >> sh) & 31
                plsc.addupdate_scatter(hist, [dig], ones16)
                return 0

            lax.fori_loop(0, trips, hcount, 0)

            h0 = hist[pl.ds(0, 16)]
            h1 = hist[pl.ds(16, 16)]
            s0, s1 = _suffix_scan(h0, h1)
            hist[pl.ds(0, 16)] = s0 - h0
            hist[pl.ds(16, 16)] = s1 - h1

            def perm(i, _c, sk=sk, si=si, dk=dk, di=di, sh=sh):
                key = sk[pl.ds(i * L, 16)]
                idxv = si[pl.ds(i * L, 16)]
                dig = (key >> sh) & 31
                cnt, last = plsc.scan_count(dig)
                basev = plsc.load_gather(hist, [dig])
                pos = basev + cnt - 1
                plsc.store_scatter(dk, [pos], key)
                plsc.store_scatter(di, [pos], idxv)
                plsc.addupdate_scatter(hist, [dig], cnt, mask=last)
                return 0

            lax.fori_loop(0, trips, perm, 0)

        def emit(t, _c):
            pos16 = t * L + iota
            sidx = ci1[pl.ds(t * L, 16)] * 2
            bb = plsc.load_gather(dgm_v, [sidx])
            dd = plsc.load_gather(dgm_v, [sidx + 1])
            plsc.store_scatter(outv, [2 * pos16], bb)
            plsc.store_scatter(outv, [2 * pos16 + 1], dd)
            return 0

        lax.fori_loop(0, K // L, emit, 0)
        pltpu.sync_copy(outv, out_hbm.at[row])
        return 0

    lax.fori_loop(0, ROWS_PER_W, do_row, 0)


def kernel(diagrams):
    return _topk_sc(diagrams.reshape(B, 2 * N))
